# baseline (speedup 1.0000x reference)
import functools

import jax
import jax.numpy as jnp
from jax import lax
from jax.experimental import pallas as pl
from jax.experimental.pallas import tpu as pltpu
from jax.experimental.pallas import tpu_sc as plsc

_S = 30.0
_LOG2E = 1.4426950408889634
_LN2 = 0.6931471805599453
_K = _S * _LOG2E
_B = 16384
_C = 100
_L = 16
_NC = 2
_NS = 16
_NW = _NC * _NS
_RW = _B // _NW
_G = _RW // _L


def _ln(s):
    bits = plsc.bitcast(s, jnp.int32)
    e = (bits >> 23) - 127
    mant = plsc.bitcast((bits & 0x7FFFFF) | 0x3F800000, jnp.float32)
    u = mant - 1.0
    y = e.astype(jnp.float32) * 0.6931472 + u * (1.0 - u * (0.5 - u * 0.33333334))
    for _ in range(3):
        y = y + s * jnp.exp(-y) - 1.0
    return y


def _sc_body(x_hbm, m_hbm, t_hbm, out_hbm, x_v, t_v, m_v, mx_v, o_v):
    wid = lax.axis_index("s") * _NC + lax.axis_index("c")
    base = wid * _RW
    pltpu.sync_copy(x_hbm.at[pl.ds(base, _RW), :], x_v)
    pltpu.sync_copy(t_hbm.at[pl.ds(base, _RW)], t_v)
    pltpu.sync_copy(m_hbm, m_v)
    lanes = lax.iota(jnp.int32, _L)

    def diag(rows, d):
        col_c = (lanes + d) % _C
        return plsc.load_gather(x_v, [rows, col_c])

    def pass1(g, carry):
        r0 = g * _L
        rows = lanes + r0

        m0 = m1 = m2 = m3 = jnp.full((_L,), -3.0e38, jnp.float32)
        for c in range(0, _C, 4):
            a = diag(rows, c)
            b = diag(rows, c + 1)
            d = diag(rows, c + 2)
            e = diag(rows, c + 3)
            m0 = jnp.maximum(m0, a)
            m1 = jnp.maximum(m1, b)
            m2 = jnp.maximum(m2, d)
            m3 = jnp.maximum(m3, e)
        mx = jnp.maximum(jnp.maximum(m0, m1), jnp.maximum(m2, m3))
        mx_v[pl.ds(r0, _L)] = mx
        return carry

    lax.fori_loop(0, _G, pass1, 0)

    def pass2(g, carry):
        r0 = g * _L
        rows = lanes + r0
        t = t_v[pl.ds(r0, _L)]
        bm = plsc.load_gather(m_v, [t])
        mx = mx_v[pl.ds(r0, _L)]

        s0 = s1 = s2 = s3 = jnp.zeros((_L,), jnp.float32)
        for c in range(0, _C, 4):
            a = diag(rows, c)
            b = diag(rows, c + 1)
            d = diag(rows, c + 2)
            e = diag(rows, c + 3)
            s0 = s0 + jnp.exp((a - mx) * _S)
            s1 = s1 + jnp.exp((b - mx) * _S)
            s2 = s2 + jnp.exp((d - mx) * _S)
            s3 = s3 + jnp.exp((e - mx) * _S)
        s = (s0 + s1) + (s2 + s3)

        xt = plsc.load_gather(x_v, [rows, t])
        e_raw = jnp.exp((xt - mx) * _S)
        e_mod = jnp.exp((xt - bm - mx) * _S)
        s = jnp.maximum(s - e_raw + e_mod, 1e-30)

        loss = _ln(s) + _S * ((mx - xt) + bm)
        o_v[pl.ds(r0, _L)] = loss
        return carry

    lax.fori_loop(0, _G, pass2, 0)
    pltpu.sync_copy(o_v, out_hbm.at[pl.ds(base, _RW)])


_sc_ldam = functools.partial(
    pl.kernel,
    out_type=jax.ShapeDtypeStruct((_B,), jnp.float32),
    mesh=plsc.VectorSubcoreMesh(
        core_axis_name="c", subcore_axis_name="s", num_cores=_NC, num_subcores=_NS
    ),
    scratch_types=[
        pltpu.VMEM((_RW, _C), jnp.float32),
        pltpu.VMEM((_RW,), jnp.int32),
        pltpu.VMEM((_C,), jnp.float32),
        pltpu.VMEM((_RW,), jnp.float32),
        pltpu.VMEM((_RW,), jnp.float32),
    ],
    compiler_params=pltpu.CompilerParams(
        needs_layout_passes=False, skip_device_barrier=True
    ),
)(_sc_body)


def _mean_body(x_ref, out_ref):
    out_ref[0, 0] = jnp.sum(x_ref[...]) * (1.0 / _B)


def kernel(inputs, m_list, targets):
    per_row = _sc_ldam(inputs, m_list, targets)
    out = pl.pallas_call(
        _mean_body,
        out_specs=pl.BlockSpec(memory_space=pltpu.SMEM),
        out_shape=jax.ShapeDtypeStruct((1, 1), jnp.float32),
    )(per_row.reshape(128, 128))
    return out[0, 0]


# speedup vs baseline: 1.0281x; 1.0281x over previous
import functools

import jax
import jax.numpy as jnp
from jax import lax
from jax.experimental import pallas as pl
from jax.experimental.pallas import tpu as pltpu
from jax.experimental.pallas import tpu_sc as plsc

_S = 30.0
_LOG2E = 1.4426950408889634
_LN2 = 0.6931471805599453
_K = _S * _LOG2E
_B = 16384
_C = 100
_L = 16
_NC = 2
_NS = 16
_NW = _NC * _NS
_BSC = 6656
_BTC = _B - _BSC
_RW = _BSC // _NW
_G = _RW // _L


def _ln(s):
    bits = plsc.bitcast(s, jnp.int32)
    e = (bits >> 23) - 127
    mant = plsc.bitcast((bits & 0x7FFFFF) | 0x3F800000, jnp.float32)
    u = mant - 1.0
    y = e.astype(jnp.float32) * 0.6931472 + u * (1.0 - u * (0.5 - u * 0.33333334))
    for _ in range(3):
        y = y + s * jnp.exp(-y) - 1.0
    return y


def _sc_body(x_hbm, m_hbm, t_hbm, out_hbm, x_v, t_v, m_v, mx_v, o_v):
    wid = lax.axis_index("s") * _NC + lax.axis_index("c")
    base = wid * _RW
    pltpu.sync_copy(x_hbm.at[pl.ds(base, _RW), :], x_v)
    pltpu.sync_copy(t_hbm.at[pl.ds(base, _RW)], t_v)
    pltpu.sync_copy(m_hbm, m_v)
    lanes = lax.iota(jnp.int32, _L)

    def diag(rows, d):
        col_c = (lanes + d) % _C
        return plsc.load_gather(x_v, [rows, col_c])

    def pass1(g, carry):
        r0 = g * _L
        rows = lanes + r0

        m0 = m1 = m2 = m3 = jnp.full((_L,), -3.0e38, jnp.float32)
        for c in range(0, _C, 4):
            a = diag(rows, c)
            b = diag(rows, c + 1)
            d = diag(rows, c + 2)
            e = diag(rows, c + 3)
            m0 = jnp.maximum(m0, a)
            m1 = jnp.maximum(m1, b)
            m2 = jnp.maximum(m2, d)
            m3 = jnp.maximum(m3, e)
        mx = jnp.maximum(jnp.maximum(m0, m1), jnp.maximum(m2, m3))
        mx_v[pl.ds(r0, _L)] = mx
        return carry

    lax.fori_loop(0, _G, pass1, 0)

    def pass2(g, carry):
        r0 = g * _L
        rows = lanes + r0
        t = t_v[pl.ds(r0, _L)]
        bm = plsc.load_gather(m_v, [t])
        mx = mx_v[pl.ds(r0, _L)]

        s0 = s1 = s2 = s3 = jnp.zeros((_L,), jnp.float32)
        for c in range(0, _C, 4):
            a = diag(rows, c)
            b = diag(rows, c + 1)
            d = diag(rows, c + 2)
            e = diag(rows, c + 3)
            s0 = s0 + jnp.exp((a - mx) * _S)
            s1 = s1 + jnp.exp((b - mx) * _S)
            s2 = s2 + jnp.exp((d - mx) * _S)
            s3 = s3 + jnp.exp((e - mx) * _S)
        s = (s0 + s1) + (s2 + s3)

        xt = plsc.load_gather(x_v, [rows, t])
        e_raw = jnp.exp((xt - mx) * _S)
        e_mod = jnp.exp((xt - bm - mx) * _S)
        s = jnp.maximum(s - e_raw + e_mod, 1e-30)

        loss = _ln(s) + _S * ((mx - xt) + bm)
        o_v[pl.ds(r0, _L)] = loss
        return carry

    lax.fori_loop(0, _G, pass2, 0)
    pltpu.sync_copy(o_v, out_hbm.at[pl.ds(base, _RW)])


_sc_ldam = functools.partial(
    pl.kernel,
    out_type=jax.ShapeDtypeStruct((_BSC,), jnp.float32),
    mesh=plsc.VectorSubcoreMesh(
        core_axis_name="c", subcore_axis_name="s", num_cores=_NC, num_subcores=_NS
    ),
    scratch_types=[
        pltpu.VMEM((_RW, _C), jnp.float32),
        pltpu.VMEM((_RW,), jnp.int32),
        pltpu.VMEM((_C,), jnp.float32),
        pltpu.VMEM((_RW,), jnp.float32),
        pltpu.VMEM((_RW,), jnp.float32),
    ],
    compiler_params=pltpu.CompilerParams(needs_layout_passes=False),
)(_sc_body)


def _tc_block(x_ref, t_ref, m_ref, out_ref):
    i = pl.program_id(0)
    x = x_ref[...]
    t = t_ref[...]
    m_row = m_ref[...]
    bm_rows, c = x.shape
    col = jax.lax.broadcasted_iota(jnp.int32, (bm_rows, c), 1)
    mask = col == t
    batch_m = jnp.sum(jnp.where(mask, m_row, 0.0), axis=1, keepdims=True)
    w = _S * (x - jnp.where(mask, batch_m, 0.0))
    mx = jnp.max(w, axis=1, keepdims=True)
    s = jnp.sum(jnp.exp(w - mx), axis=1)
    wt = jnp.sum(jnp.where(mask, w, 0.0), axis=1)
    blk = jnp.sum(jnp.log(s) + mx[:, 0] - wt)

    @pl.when(i == 0)
    def _init():
        out_ref[0, 0] = 0.0

    out_ref[0, 0] += blk


def _finish_body(sc_ref, tcs_ref, out_ref):
    out_ref[0, 0] = (jnp.sum(sc_ref[...]) + tcs_ref[0, 0]) * (1.0 / _B)


def kernel(inputs, m_list, targets):
    n, c = inputs.shape
    sc_rows = _sc_ldam(inputs[:_BSC], m_list, targets[:_BSC])
    bm_rows = 1216
    tc_sum = pl.pallas_call(
        _tc_block,
        grid=(_BTC // bm_rows,),
        in_specs=[
            pl.BlockSpec((bm_rows, c), lambda i: (i, 0)),
            pl.BlockSpec((bm_rows, 1), lambda i: (i, 0)),
            pl.BlockSpec((1, c), lambda i: (0, 0)),
        ],
        out_specs=pl.BlockSpec((1, 1), lambda i: (0, 0), memory_space=pltpu.SMEM),
        out_shape=jax.ShapeDtypeStruct((1, 1), jnp.float32),
    )(inputs[_BSC:], targets[_BSC:].reshape(_BTC, 1), m_list.reshape(1, c))
    out = pl.pallas_call(
        _finish_body,
        in_specs=[
            pl.BlockSpec(memory_space=pltpu.VMEM),
            pl.BlockSpec(memory_space=pltpu.SMEM),
        ],
        out_specs=pl.BlockSpec(memory_space=pltpu.SMEM),
        out_shape=jax.ShapeDtypeStruct((1, 1), jnp.float32),
    )(sc_rows.reshape(_BSC // 128, 128), tc_sum)
    return out[0, 0]


# speedup vs baseline: 1.6927x; 1.6464x over previous
import functools

import jax
import jax.numpy as jnp
from jax import lax
from jax.experimental import pallas as pl
from jax.experimental.pallas import tpu as pltpu
from jax.experimental.pallas import tpu_sc as plsc

_S = 30.0
_B = 16384
_C = 100
_L = 16
_NS = 16
_RW = _B // _NS
_G = _RW // _L
_BM = 1024
_NBT = _B // _BM


def _sc_body(xt_hbm, m_hbm, t_hbm, xt_out, bm_out, x_v, t_v, m_v, xo_v, bo_v):
    wid = lax.axis_index("s")
    base = wid * _RW
    pltpu.sync_copy(xt_hbm.at[:, pl.ds(base, _RW)], x_v)
    pltpu.sync_copy(t_hbm.at[pl.ds(base, _RW)], t_v)
    pltpu.sync_copy(m_hbm, m_v)
    lanes = lax.iota(jnp.int32, _L)

    def group(g, carry):
        r0 = g * _L
        rows = lanes + r0
        t = t_v[pl.ds(r0, _L)]
        bo_v[pl.ds(r0, _L)] = plsc.load_gather(m_v, [t])
        xo_v[pl.ds(r0, _L)] = plsc.load_gather(x_v, [t, rows])
        return carry

    lax.fori_loop(0, _G, group, 0)
    pltpu.sync_copy(xo_v, xt_out.at[wid, 0])
    pltpu.sync_copy(bo_v, bm_out.at[wid, 0])


_sc_gather = functools.partial(
    pl.kernel,
    out_type=[
        jax.ShapeDtypeStruct((_NS, 1, _RW), jnp.float32),
        jax.ShapeDtypeStruct((_NS, 1, _RW), jnp.float32),
    ],
    mesh=plsc.VectorSubcoreMesh(
        core_axis_name="c", subcore_axis_name="s", num_cores=1, num_subcores=_NS
    ),
    scratch_types=[
        pltpu.VMEM((_C, _RW), jnp.float32),
        pltpu.VMEM((_RW,), jnp.int32),
        pltpu.VMEM((_C,), jnp.float32),
        pltpu.VMEM((_RW,), jnp.float32),
        pltpu.VMEM((_RW,), jnp.float32),
    ],
    compiler_params=pltpu.CompilerParams(needs_layout_passes=False),
)(_sc_body)


def _tc_block(xt_ref, s_ref, mx_ref):
    x = xt_ref[...]
    mx = jnp.max(x, axis=0, keepdims=True)
    s = jnp.sum(jnp.exp((x - mx) * _S), axis=0, keepdims=True)
    s_ref[0] = s
    mx_ref[0] = mx


def _combine_body(s_ref, mx_ref, xt_ref, bm_ref, out_ref):
    s = s_ref[...]
    mx = mx_ref[...]
    xt = xt_ref[...]
    bm = bm_ref[...]
    e_raw = jnp.exp((xt - mx) * _S)
    e_mod = jnp.exp((xt - bm - mx) * _S)
    s2 = jnp.maximum(s - e_raw + e_mod, 1e-30)
    loss = jnp.log(s2) + _S * ((mx - xt) + bm)
    out_ref[0, 0] = jnp.sum(loss) * (1.0 / _B)


def kernel(inputs, m_list, targets):
    xt = inputs.T
    xt_a, bm_a = _sc_gather(xt, m_list, targets)
    s_a, mx_a = pl.pallas_call(
        _tc_block,
        grid=(_NBT,),
        in_specs=[pl.BlockSpec((_C, _BM), lambda i: (0, i))],
        out_specs=[
            pl.BlockSpec((1, 1, _BM), lambda i: (i, 0, 0)),
            pl.BlockSpec((1, 1, _BM), lambda i: (i, 0, 0)),
        ],
        out_shape=[
            jax.ShapeDtypeStruct((_NBT, 1, _BM), jnp.float32),
            jax.ShapeDtypeStruct((_NBT, 1, _BM), jnp.float32),
        ],
    )(xt)
    out = pl.pallas_call(
        _combine_body,
        in_specs=[
            pl.BlockSpec(memory_space=pltpu.VMEM),
            pl.BlockSpec(memory_space=pltpu.VMEM),
            pl.BlockSpec(memory_space=pltpu.VMEM),
            pl.BlockSpec(memory_space=pltpu.VMEM),
        ],
        out_specs=pl.BlockSpec(memory_space=pltpu.SMEM),
        out_shape=jax.ShapeDtypeStruct((1, 1), jnp.float32),
    )(s_a, mx_a, xt_a, bm_a)
    return out[0, 0]


# speedup vs baseline: 1.7311x; 1.0227x over previous
import functools

import jax
import jax.numpy as jnp
from jax import lax
from jax.experimental import pallas as pl
from jax.experimental.pallas import tpu as pltpu
from jax.experimental.pallas import tpu_sc as plsc

_S = 30.0
_B = 16384
_C = 100
_L = 16
_NS = 16
_RW = _B // _NS
_G = _RW // _L
_BM = 2048
_NBT = _B // _BM


def _sc_body(xt_hbm, m_hbm, t_hbm, xt_out, bm_out, x_v, t_v, m_v, xo_v, bo_v):
    wid = lax.axis_index("s")
    base = wid * _RW
    pltpu.sync_copy(xt_hbm.at[:, pl.ds(base, _RW)], x_v)
    pltpu.sync_copy(t_hbm.at[pl.ds(base, _RW)], t_v)
    pltpu.sync_copy(m_hbm, m_v)
    lanes = lax.iota(jnp.int32, _L)

    def group(g, carry):
        r0 = g * _L
        rows = lanes + r0
        t = t_v[pl.ds(r0, _L)]
        bo_v[pl.ds(r0, _L)] = plsc.load_gather(m_v, [t])
        xo_v[pl.ds(r0, _L)] = plsc.load_gather(x_v, [t, rows])
        return carry

    lax.fori_loop(0, _G, group, 0)
    blk = wid // (_BM // _RW)
    off = (wid % (_BM // _RW)) * _RW
    pltpu.sync_copy(xo_v, xt_out.at[blk, 0, pl.ds(off, _RW)])
    pltpu.sync_copy(bo_v, bm_out.at[blk, 0, pl.ds(off, _RW)])


_sc_gather = functools.partial(
    pl.kernel,
    out_type=[
        jax.ShapeDtypeStruct((_NBT, 1, _BM), jnp.float32),
        jax.ShapeDtypeStruct((_NBT, 1, _BM), jnp.float32),
    ],
    mesh=plsc.VectorSubcoreMesh(
        core_axis_name="c", subcore_axis_name="s", num_cores=1, num_subcores=_NS
    ),
    scratch_types=[
        pltpu.VMEM((_C, _RW), jnp.float32),
        pltpu.VMEM((_RW,), jnp.int32),
        pltpu.VMEM((_C,), jnp.float32),
        pltpu.VMEM((_RW,), jnp.float32),
        pltpu.VMEM((_RW,), jnp.float32),
    ],
    compiler_params=pltpu.CompilerParams(needs_layout_passes=False),
)(_sc_body)


def _tc_block(xt_ref, s_ref, mx_ref):
    x = xt_ref[...]
    mx = jnp.max(x, axis=0, keepdims=True)
    s = jnp.sum(jnp.exp((x - mx) * _S), axis=0, keepdims=True)
    s_ref[0] = s
    mx_ref[0] = mx


def _combine_body(s_ref, mx_ref, xt_ref, bm_ref, out_ref):
    s = s_ref[...]
    mx = mx_ref[...]
    xt = xt_ref[...]
    bm = bm_ref[...]
    e_raw = jnp.exp((xt - mx) * _S)
    e_mod = jnp.exp((xt - bm - mx) * _S)
    s2 = jnp.maximum(s - e_raw + e_mod, 1e-30)
    loss = jnp.log(s2) + _S * ((mx - xt) + bm)
    out_ref[0, 0] = jnp.sum(loss) * (1.0 / _B)


def kernel(inputs, m_list, targets):
    xt = inputs.T
    xt_a, bm_a = _sc_gather(xt, m_list, targets)
    s_a, mx_a = pl.pallas_call(
        _tc_block,
        grid=(_NBT,),
        in_specs=[pl.BlockSpec((_C, _BM), lambda i: (0, i))],
        out_specs=[
            pl.BlockSpec((1, 1, _BM), lambda i: (i, 0, 0)),
            pl.BlockSpec((1, 1, _BM), lambda i: (i, 0, 0)),
        ],
        out_shape=[
            jax.ShapeDtypeStruct((_NBT, 1, _BM), jnp.float32),
            jax.ShapeDtypeStruct((_NBT, 1, _BM), jnp.float32),
        ],
    )(xt)
    out = pl.pallas_call(
        _combine_body,
        in_specs=[
            pl.BlockSpec(memory_space=pltpu.VMEM),
            pl.BlockSpec(memory_space=pltpu.VMEM),
            pl.BlockSpec(memory_space=pltpu.VMEM),
            pl.BlockSpec(memory_space=pltpu.VMEM),
        ],
        out_specs=pl.BlockSpec(memory_space=pltpu.SMEM),
        out_shape=jax.ShapeDtypeStruct((1, 1), jnp.float32),
    )(s_a, mx_a, xt_a, bm_a)
    return out[0, 0]
